# baseline (speedup 1.0000x reference)
import functools

import jax
import jax.numpy as jnp
from jax import lax
from jax.experimental import pallas as pl
from jax.experimental.pallas import tpu as pltpu
from jax.experimental.pallas import tpu_sc as plsc

NC, NS = 2, 16
NW = NC * NS
B = 16384
D = 64
DP = 2 * D
BPW = B // NW
CH = 128
NCH = BPW // CH
LANES = 16
NG = CH // LANES


def _gather_u_body(uh_hbm, uw_hbm, up_hbm, uhalf, buf0, buf1, sem):
    wid = lax.axis_index("s") * NC + lax.axis_index("c")
    pltpu.sync_copy(uh_hbm.at[wid], uhalf)
    bufs = (buf0, buf1)
    cps = [None] * NCH
    for c in range(min(2, NCH)):
        cps[c] = pltpu.async_copy(uw_hbm.at[uhalf.at[c]], bufs[c % 2], sem)
    for c in range(NCH):
        cps[c].wait()
        pltpu.sync_copy(bufs[c % 2],
                        up_hbm.at[pl.ds(wid * BPW + c * CH, CH)])
        if c + 2 < NCH:
            cps[c + 2] = pltpu.async_copy(
                uw_hbm.at[uhalf.at[c + 2]], bufs[c % 2], sem)


def _dot_body(ih_hbm, jh_hbm, u_hbm, i_hbm, j_hbm, up_hbm, iw_hbm,
              out_i_hbm, out_j_hbm,
              ihalf, jhalf, uidx, iidx, jidx,
              urows, irows, jrows, oi, oj, sem):
    wid = lax.axis_index("s") * NC + lax.axis_index("c")
    pltpu.sync_copy(ih_hbm.at[wid], ihalf)
    pltpu.sync_copy(jh_hbm.at[wid], jhalf)
    pltpu.sync_copy(u_hbm.at[wid], uidx)
    pltpu.sync_copy(i_hbm.at[wid], iidx)
    pltpu.sync_copy(j_hbm.at[wid], jidx)

    iota = lax.iota(jnp.int32, LANES)

    def chunk(c, carry):
        cps = [
            pltpu.async_copy(
                up_hbm.at[pl.ds(wid * BPW + c * CH, CH)], urows, sem),
            pltpu.async_copy(iw_hbm.at[ihalf.at[c]], irows, sem),
            pltpu.async_copy(iw_hbm.at[jhalf.at[c]], jrows, sem),
        ]
        for cp in cps:
            cp.wait()
        out_base = c * CH

        def group(g, carry2):
            s = g * LANES
            rows = s + iota
            u_col = (uidx[c, pl.ds(s, LANES)] & 1) * D
            i_col = (iidx[c, pl.ds(s, LANES)] & 1) * D
            j_col = (jidx[c, pl.ds(s, LANES)] & 1) * D
            u0 = plsc.load_gather(urows, [rows, u_col])
            acc_i = u0 * plsc.load_gather(irows, [rows, i_col])
            acc_j = u0 * plsc.load_gather(jrows, [rows, j_col])
            for d in range(1, D):
                ud = plsc.load_gather(urows, [rows, u_col + d])
                acc_i = acc_i + ud * plsc.load_gather(irows, [rows, i_col + d])
                acc_j = acc_j + ud * plsc.load_gather(jrows, [rows, j_col + d])
            oi[pl.ds(out_base + s, LANES)] = acc_i
            oj[pl.ds(out_base + s, LANES)] = acc_j
            return carry2

        lax.fori_loop(0, NG, group, 0)
        return carry

    lax.fori_loop(0, NCH, chunk, 0)

    pltpu.sync_copy(oi, out_i_hbm.at[pl.ds(wid * BPW, BPW)])
    pltpu.sync_copy(oj, out_j_hbm.at[pl.ds(wid * BPW, BPW)])


@jax.jit
def _sc_bpr(uh3, ih3, jh3, user3, ii3, ij3, uw2, iw2):
    f32 = jnp.float32
    mesh = plsc.VectorSubcoreMesh(
        core_axis_name="c", subcore_axis_name="s",
        num_cores=NC, num_subcores=NS)
    params = pltpu.CompilerParams(needs_layout_passes=False)

    gather_u = pl.kernel(
        _gather_u_body,
        out_type=jax.ShapeDtypeStruct((B, DP), f32),
        mesh=mesh,
        scratch_types=[
            pltpu.VMEM((NCH, CH), jnp.int32),
            pltpu.VMEM((CH, DP), f32),
            pltpu.VMEM((CH, DP), f32),
            pltpu.SemaphoreType.DMA,
        ],
        compiler_params=params,
    )
    upairs = gather_u(uh3, uw2)

    dot = pl.kernel(
        _dot_body,
        out_type=(jax.ShapeDtypeStruct((B,), f32),
                  jax.ShapeDtypeStruct((B,), f32)),
        mesh=mesh,
        scratch_types=[
            pltpu.VMEM((NCH, CH), jnp.int32),
            pltpu.VMEM((NCH, CH), jnp.int32),
            pltpu.VMEM((NCH, CH), jnp.int32),
            pltpu.VMEM((NCH, CH), jnp.int32),
            pltpu.VMEM((NCH, CH), jnp.int32),
            pltpu.VMEM((CH, DP), f32),
            pltpu.VMEM((CH, DP), f32),
            pltpu.VMEM((CH, DP), f32),
            pltpu.VMEM((BPW,), f32),
            pltpu.VMEM((BPW,), f32),
            pltpu.SemaphoreType.DMA,
        ],
        compiler_params=params,
    )
    return dot(ih3, jh3, user3, ii3, ij3, upairs, iw2)


def kernel(user, item_i, item_j, embed_user_w, embed_item_w):
    user = user.astype(jnp.int32)
    item_i = item_i.astype(jnp.int32)
    item_j = item_j.astype(jnp.int32)
    uh3 = (user >> 1).reshape(NW, NCH, CH)
    ih3 = (item_i >> 1).reshape(NW, NCH, CH)
    jh3 = (item_j >> 1).reshape(NW, NCH, CH)
    user3 = user.reshape(NW, NCH, CH)
    ii3 = item_i.reshape(NW, NCH, CH)
    ij3 = item_j.reshape(NW, NCH, CH)
    uw2 = embed_user_w.reshape(-1, DP)
    iw2 = embed_item_w.reshape(-1, DP)
    return _sc_bpr(uh3, ih3, jh3, user3, ii3, ij3, uw2, iw2)
